# Initial kernel scaffold; baseline (speedup 1.0000x reference)
import jax
import jax.numpy as jnp
from jax.experimental import pallas as pl


def kernel(x, tables):
    raise NotImplementedError("write your pallas kernel here")


# speedup vs baseline: 1.0139x; 1.0139x over previous
import functools

import jax
import jax.numpy as jnp
from jax import lax
from jax.experimental import pallas as pl
from jax.experimental.pallas import tpu as pltpu
from jax.experimental.pallas import tpu_sc as plsc

_NUM_FIELDS = 26
_VOCAB = 100000
_EMB_DIM = 32
_BATCH = 4096

_NC = 2
_NS = 16
_NW = _NC * _NS

_TOTAL = _BATCH * _NUM_FIELDS
_BPW = _TOTAL // _NW
_LANES = 16
_NVEC = _BPW // _LANES
_GCHUNK = 128
_NGATHER = _BPW // _GCHUNK

_mesh = plsc.VectorSubcoreMesh(core_axis_name="c", subcore_axis_name="s")


@functools.partial(
    pl.kernel,
    out_type=jax.ShapeDtypeStruct((_TOTAL, _EMB_DIM), jnp.float32),
    mesh=_mesh,
    scratch_types=[
        pltpu.VMEM((_BPW,), jnp.int32),
        pltpu.VMEM((_BPW, _EMB_DIM), jnp.float32),
        pltpu.SemaphoreType.DMA,
    ],
    compiler_params=pltpu.CompilerParams(use_tc_tiling_on_sc=False),
)
def _gather_all(ids_hbm, tab_hbm, out_hbm, idx_v, rows_v, sem):
    wid = lax.axis_index("s") * _NC + lax.axis_index("c")
    base = wid * _BPW

    pltpu.sync_copy(ids_hbm.at[pl.ds(base, _BPW)], idx_v)

    def add_offsets(p, c):
        sl = pl.ds(pl.multiple_of(p * _LANES, _LANES), _LANES)
        idx_v[sl] = idx_v[sl] + c * _VOCAB
        c = c + _LANES
        return jnp.where(c >= _NUM_FIELDS, c - _NUM_FIELDS, c)

    lax.fori_loop(0, _NVEC, add_offsets, lax.iota(jnp.int32, _LANES))

    def fire(k, _):
        off = pl.multiple_of(k * _GCHUNK, _GCHUNK)
        pltpu.make_async_copy(
            tab_hbm.at[idx_v.at[pl.ds(off, _GCHUNK)]],
            rows_v.at[pl.ds(off, _GCHUNK)],
            sem,
        ).start()
        return 0

    lax.fori_loop(0, _NGATHER, fire, 0)
    pltpu.make_async_copy(tab_hbm.at[pl.ds(0, _BPW)], rows_v, sem).wait()

    pltpu.sync_copy(rows_v, out_hbm.at[pl.ds(base, _BPW)])


def kernel(x, tables):
    ids = x.reshape(_TOTAL)
    tab = tables.reshape(_NUM_FIELDS * _VOCAB, _EMB_DIM)
    out = _gather_all(ids, tab)
    return out.reshape(_BATCH, _NUM_FIELDS, _EMB_DIM)


# speedup vs baseline: 2.0202x; 1.9926x over previous
import functools

import jax
import jax.numpy as jnp
from jax import lax
from jax.experimental import pallas as pl
from jax.experimental.pallas import tpu as pltpu
from jax.experimental.pallas import tpu_sc as plsc

_NUM_FIELDS = 26
_VOCAB = 100000
_EMB_DIM = 32
_BATCH = 4096

_NC = 2
_NS = 16
_NW = _NC * _NS

_ROWS = _NUM_FIELDS * _EMB_DIM
_RPW = _ROWS // _NW

_mesh = plsc.VectorSubcoreMesh(core_axis_name="c", subcore_axis_name="s")


@functools.partial(
    pl.kernel,
    out_type=jax.ShapeDtypeStruct((_ROWS * _BATCH,), jnp.float32),
    mesh=_mesh,
    scratch_types=[
        pltpu.VMEM((2 * _BATCH,), jnp.int32),
        pltpu.VMEM((_RPW * _BATCH,), jnp.float32),
        pltpu.SemaphoreType.DMA,
    ],
    compiler_params=pltpu.CompilerParams(use_tc_tiling_on_sc=False),
)
def _gather_all(ids_hbm, tab_hbm, out_hbm, ids_v, rows_v, sem):
    wid = lax.axis_index("s") * _NC + lax.axis_index("c")
    r0 = wid * _RPW
    f0 = r0 // _EMB_DIM
    f1 = jnp.minimum(f0 + 1, _NUM_FIELDS - 1)

    pltpu.sync_copy(ids_hbm.at[pl.ds(f0 * _BATCH, _BATCH)],
                    ids_v.at[pl.ds(0, _BATCH)])
    pltpu.sync_copy(ids_hbm.at[pl.ds(f1 * _BATCH, _BATCH)],
                    ids_v.at[pl.ds(_BATCH, _BATCH)])

    def copy_q(q):
        r = r0 + q
        lf = r // _EMB_DIM - f0
        return pltpu.make_async_copy(
            tab_hbm.at[r].at[ids_v.at[pl.ds(lf * _BATCH, _BATCH)]],
            rows_v.at[pl.ds(q * _BATCH, _BATCH)],
            sem,
        )

    def fire(q, _):
        copy_q(q).start()
        return 0

    def drain(q, _):
        copy_q(q).wait()
        return 0

    lax.fori_loop(0, _RPW, fire, 0)
    lax.fori_loop(0, _RPW, drain, 0)

    pltpu.sync_copy(rows_v, out_hbm.at[pl.ds(r0 * _BATCH, _RPW * _BATCH)])


def kernel(x, tables):
    ids = x.T.reshape(_NUM_FIELDS * _BATCH)
    tab2 = tables.transpose(0, 2, 1).reshape(_ROWS, _VOCAB)
    out = _gather_all(ids, tab2)
    return out.reshape(_NUM_FIELDS, _EMB_DIM, _BATCH).transpose(2, 0, 1)


# speedup vs baseline: 2.8589x; 1.4151x over previous
import functools

import jax
import jax.numpy as jnp
from jax import lax
from jax.experimental import pallas as pl
from jax.experimental.pallas import tpu as pltpu
from jax.experimental.pallas import tpu_sc as plsc

_NUM_FIELDS = 26
_VOCAB = 100000
_EMB_DIM = 32
_BATCH = 4096

_NC = 2
_NS = 16
_NW = _NC * _NS

_ROWS = _NUM_FIELDS * _EMB_DIM
_RPW = _ROWS // _NW

_NT = 100096 // 128
_GROUPS = _ROWS // 8
_GWORDS = _NT * 1024
_FLAT = _GROUPS * _NT * 8
_SLICE = 799872
_CT = 391
_CSPLIT = _NT // _CT

_mesh = plsc.VectorSubcoreMesh(core_axis_name="c", subcore_axis_name="s")


def _relabel_body(in_ref, out_ref):
    blk = in_ref[...]
    out_ref[...] = (
        blk.reshape(8, _CT, 128).swapaxes(0, 1).reshape(_CT * 8, 128)
    )


_relabel = pl.pallas_call(
    _relabel_body,
    out_shape=jax.ShapeDtypeStruct((_FLAT, 128), jnp.float32),
    grid=(_GROUPS, _CSPLIT),
    in_specs=[pl.BlockSpec((8, _CT * 128), lambda g, c: (g, c))],
    out_specs=pl.BlockSpec((_CT * 8, 128), lambda g, c: (g * _CSPLIT + c, 0)),
)


@functools.partial(
    pl.kernel,
    out_type=jax.ShapeDtypeStruct((_ROWS * _BATCH,), jnp.float32),
    mesh=_mesh,
    scratch_types=[
        pltpu.VMEM((2 * _BATCH,), jnp.int32),
        pltpu.VMEM((_RPW * _BATCH,), jnp.float32),
        pltpu.SemaphoreType.DMA,
    ],
    compiler_params=pltpu.CompilerParams(use_tc_tiling_on_sc=False),
)
def _gather_all(ids_hbm, flat_hbm, out_hbm, offs_v, rows_v, sem):
    wid = lax.axis_index("s") * _NC + lax.axis_index("c")
    r0 = wid * _RPW
    f0 = r0 // _EMB_DIM
    f1 = jnp.minimum(f0 + 1, _NUM_FIELDS - 1)

    pltpu.sync_copy(ids_hbm.at[pl.ds(f0 * _BATCH, _BATCH)],
                    offs_v.at[pl.ds(0, _BATCH)])
    pltpu.sync_copy(ids_hbm.at[pl.ds(f1 * _BATCH, _BATCH)],
                    offs_v.at[pl.ds(_BATCH, _BATCH)])

    def to_offs(p, _):
        sl = pl.ds(pl.multiple_of(p * 16, 16), 16)
        v = offs_v[sl]
        offs_v[sl] = ((v >> 7) << 10) | (v & 127)
        return 0

    lax.fori_loop(0, 2 * _BATCH // 16, to_offs, 0)

    def copy_q(q):
        r = r0 + q
        lf = r // _EMB_DIM - f0
        base = pl.multiple_of((r // 8) * _GWORDS + (r % 8) * 128, 128)
        return pltpu.make_async_copy(
            flat_hbm.at[pl.ds(base, _SLICE)]
                    .at[offs_v.at[pl.ds(lf * _BATCH, _BATCH)]],
            rows_v.at[pl.ds(q * _BATCH, _BATCH)],
            sem,
        )

    def fire(q, _):
        copy_q(q).start()
        return 0

    def drain(q, _):
        copy_q(q).wait()
        return 0

    lax.fori_loop(0, _RPW, fire, 0)
    lax.fori_loop(0, _RPW, drain, 0)

    pltpu.sync_copy(rows_v, out_hbm.at[pl.ds(r0 * _BATCH, _RPW * _BATCH)])


def kernel(x, tables):
    ids = x.T.reshape(_NUM_FIELDS * _BATCH)
    tab2 = tables.transpose(0, 2, 1).reshape(_ROWS, _VOCAB)
    flat = _relabel(tab2).reshape(_FLAT * 128)
    out = _gather_all(ids, flat)
    return out.reshape(_NUM_FIELDS, _EMB_DIM, _BATCH).transpose(2, 0, 1)


# speedup vs baseline: 3.2856x; 1.1493x over previous
import functools

import jax
import jax.numpy as jnp
from jax import lax
from jax.experimental import pallas as pl
from jax.experimental.pallas import tpu as pltpu
from jax.experimental.pallas import tpu_sc as plsc

_NUM_FIELDS = 26
_VOCAB = 100000
_EMB_DIM = 32
_BATCH = 4096

_NC = 2
_NS = 16
_NW = _NC * _NS

_ROWS = _NUM_FIELDS * _EMB_DIM
_RPW = _ROWS // _NW

_NT = 100096 // 128
_GROUPS = _ROWS // 8
_GWORDS = _NT * 1024
_FLAT = _GROUPS * _NT * 8
_SLICE = 799872
_CT = 782
_CSPLIT = _NT // _CT

_mesh = plsc.VectorSubcoreMesh(core_axis_name="c", subcore_axis_name="s")


def _relabel_body(in_ref, out_ref):
    blk = in_ref[...]
    out_ref[...] = (
        blk.reshape(8, _CT, 128).swapaxes(0, 1).reshape(_CT * 8, 128)
    )


_relabel = pl.pallas_call(
    _relabel_body,
    out_shape=jax.ShapeDtypeStruct((_FLAT, 128), jnp.float32),
    grid=(_GROUPS, _CSPLIT),
    in_specs=[pl.BlockSpec((8, _CT * 128), lambda g, c: (g, c))],
    out_specs=pl.BlockSpec((_CT * 8, 128), lambda g, c: (g * _CSPLIT + c, 0)),
)


@functools.partial(
    pl.kernel,
    out_type=jax.ShapeDtypeStruct((_ROWS * _BATCH,), jnp.float32),
    mesh=_mesh,
    scratch_types=[
        pltpu.VMEM((2 * _BATCH,), jnp.int32),
        pltpu.VMEM((_RPW * _BATCH,), jnp.float32),
        pltpu.SemaphoreType.DMA,
    ],
    compiler_params=pltpu.CompilerParams(use_tc_tiling_on_sc=False),
)
def _gather_all(ids_hbm, flat_hbm, out_hbm, offs_v, rows_v, sem):
    wid = lax.axis_index("s") * _NC + lax.axis_index("c")
    r0 = wid * _RPW
    f0 = r0 // _EMB_DIM
    f1 = jnp.minimum(f0 + 1, _NUM_FIELDS - 1)

    pltpu.sync_copy(ids_hbm.at[pl.ds(f0 * _BATCH, _BATCH)],
                    offs_v.at[pl.ds(0, _BATCH)])
    pltpu.sync_copy(ids_hbm.at[pl.ds(f1 * _BATCH, _BATCH)],
                    offs_v.at[pl.ds(_BATCH, _BATCH)])

    def to_offs(p, _):
        sl = pl.ds(pl.multiple_of(p * 16, 16), 16)
        v = offs_v[sl]
        offs_v[sl] = ((v >> 7) << 10) | (v & 127)
        return 0

    lax.fori_loop(0, 2 * _BATCH // 16, to_offs, 0)

    def copy_q(q):
        r = r0 + q
        lf = r // _EMB_DIM - f0
        base = pl.multiple_of((r // 8) * _GWORDS + (r % 8) * 128, 128)
        return pltpu.make_async_copy(
            flat_hbm.at[pl.ds(base, _SLICE)]
                    .at[offs_v.at[pl.ds(lf * _BATCH, _BATCH)]],
            rows_v.at[pl.ds(q * _BATCH, _BATCH)],
            sem,
        )

    def fire(q, _):
        copy_q(q).start()
        return 0

    def drain(q, _):
        copy_q(q).wait()
        return 0

    lax.fori_loop(0, _RPW, fire, 0)
    lax.fori_loop(0, _RPW, drain, 0)

    pltpu.sync_copy(rows_v, out_hbm.at[pl.ds(r0 * _BATCH, _RPW * _BATCH)])


def kernel(x, tables):
    ids = x.T.reshape(_NUM_FIELDS * _BATCH)
    tab2 = tables.transpose(0, 2, 1).reshape(_ROWS, _VOCAB)
    flat = _relabel(tab2).reshape(_FLAT * 128)
    out = _gather_all(ids, flat)
    return out.reshape(_NUM_FIELDS, _EMB_DIM, _BATCH).transpose(2, 0, 1)
